# baseline (speedup 1.0000x reference)
import functools

import jax
import jax.numpy as jnp
from jax.experimental import pallas as pl
from jax.experimental.pallas import tpu as pltpu

F = 128
BR1 = 320
BR = 1280
_PAD = 1280
ASCALE = 8192.0
F8 = jnp.float8_e4m3fn


def _rows_lt(n, base_rows, shape):
    rows = base_rows + jax.lax.broadcasted_iota(jnp.int32, shape, 0)
    return rows < n


def _support0_body(n, x_ref, w_ref, o_ref):
    r = pl.program_id(0)
    s = jnp.dot(x_ref[...].astype(jnp.bfloat16), w_ref[...],
                preferred_element_type=jnp.float32)
    s = jnp.where(_rows_lt(n, r * o_ref.shape[0], s.shape), s, 0.0)
    o_ref[...] = s.astype(jnp.bfloat16)


def _epilogue(acc, b_ref, w_ref, rowmask):
    h = jnp.maximum(acc + b_ref[...], 0.0)
    s2 = jnp.dot(h.astype(jnp.bfloat16), w_ref[...],
                 preferred_element_type=jnp.float32)
    return jnp.where(rowmask, s2, 0.0).astype(jnp.bfloat16)


def _layer1_body(n, adj_ref, s_ref, w_ref, b_ref, ones_ref,
                 aq_ref, o_ref, ocrow_ref):
    r = pl.program_id(0)
    a = adj_ref[...]
    rows = _rows_lt(n, r * BR1, a.shape)
    cols = jax.lax.broadcasted_iota(jnp.int32, a.shape, 1) < n
    am = jnp.where(rows & cols, a, 0.0)
    aq = (am * ASCALE).astype(F8)
    aq_ref[...] = aq
    a16 = aq.astype(jnp.bfloat16)
    rs = jnp.dot(a16, ones_ref[...], preferred_element_type=jnp.float32
                 )[:, 0:1]
    ocrow_ref[...] = jnp.where(rs > 0.0, 1.0 / rs, 0.0)
    acc = jnp.dot(am.astype(jnp.bfloat16), s_ref[...],
                  preferred_element_type=jnp.float32)
    o_ref[...] = _epilogue(acc, b_ref, w_ref,
                           _rows_lt(n, r * BR1, (BR1, F)))


def _mid_body(n, aq_ref, s_ref, crow_ref, w_ref, b_ref, o_ref):
    r = pl.program_id(0)
    a16 = aq_ref[...].astype(jnp.bfloat16)
    acc = jnp.dot(a16, s_ref[...], preferred_element_type=jnp.float32)
    acc = acc * crow_ref[...]
    o_ref[...] = _epilogue(acc, b_ref, w_ref, _rows_lt(n, r * BR, (BR, F)))


def _last_body(aq_ref, s_ref, crow_ref, b_ref, o_ref):
    a16 = aq_ref[...].astype(jnp.bfloat16)
    acc = jnp.dot(a16, s_ref[...], preferred_element_type=jnp.float32)
    o_ref[...] = acc * crow_ref[...] + b_ref[...]


def _cparams():
    return pltpu.CompilerParams(dimension_semantics=("arbitrary",))


def kernel(x, adj, W1, b1, W2, b2, W3, b3, W4, b4, W5, b5, W6, b6):
    n = x.shape[0]
    np_ = ((n + _PAD - 1) // _PAD) * _PAD
    f32 = jnp.float32
    bf16 = jnp.bfloat16
    gr1, gr = np_ // BR1, np_ // BR
    w16 = [w.astype(bf16) for w in (W1, W2, W3, W4, W5, W6)]
    bs = [b.reshape(1, F) for b in (b1, b2, b3, b4, b5, b6)]

    full_s = pl.BlockSpec((np_, F), lambda r: (0, 0))
    full_w = pl.BlockSpec((F, F), lambda r: (0, 0))
    full_b = pl.BlockSpec((1, F), lambda r: (0, 0))

    s = pl.pallas_call(
        functools.partial(_support0_body, n),
        grid=(gr,),
        in_specs=[pl.BlockSpec((BR, F), lambda r: (r, 0)), full_w],
        out_specs=pl.BlockSpec((BR, F), lambda r: (r, 0)),
        out_shape=jax.ShapeDtypeStruct((np_, F), bf16),
        compiler_params=_cparams(),
    )(x, w16[0])

    ones16 = jnp.ones((np_, F), bf16)
    adjq, s, crow = pl.pallas_call(
        functools.partial(_layer1_body, n),
        grid=(gr1,),
        in_specs=[pl.BlockSpec((BR1, np_), lambda r: (r, 0)),
                  full_s, full_w, full_b, full_s],
        out_specs=[pl.BlockSpec((BR1, np_), lambda r: (r, 0)),
                   pl.BlockSpec((BR1, F), lambda r: (r, 0)),
                   pl.BlockSpec((BR1, 1), lambda r: (r, 0))],
        out_shape=[jax.ShapeDtypeStruct((np_, np_), F8),
                   jax.ShapeDtypeStruct((np_, F), bf16),
                   jax.ShapeDtypeStruct((np_, 1), f32)],
        compiler_params=_cparams(),
    )(adj, s, w16[1], bs[0], ones16)

    crow_spec = pl.BlockSpec((BR, 1), lambda r: (r, 0))
    for li in (1, 2, 3, 4):
        s = pl.pallas_call(
            functools.partial(_mid_body, n),
            grid=(gr,),
            in_specs=[pl.BlockSpec((BR, np_), lambda r: (r, 0)),
                      full_s, crow_spec, full_w, full_b],
            out_specs=pl.BlockSpec((BR, F), lambda r: (r, 0)),
            out_shape=jax.ShapeDtypeStruct((np_, F), bf16),
            compiler_params=_cparams(),
        )(adjq, s, crow, w16[li + 1], bs[li])

    out = pl.pallas_call(
        _last_body,
        grid=(gr,),
        in_specs=[pl.BlockSpec((BR, np_), lambda r: (r, 0)),
                  full_s, crow_spec, full_b],
        out_specs=pl.BlockSpec((BR, F), lambda r: (r, 0)),
        out_shape=jax.ShapeDtypeStruct((np_, F), f32),
        compiler_params=_cparams(),
    )(adjq, s, crow, bs[5])

    return out[:n]


# speedup vs baseline: 1.0746x; 1.0746x over previous
import functools

import jax
import jax.numpy as jnp
from jax.experimental import pallas as pl
from jax.experimental.pallas import tpu as pltpu

F = 128
BR1 = 320
BR = 640
_PAD = 640
ASCALE = 8192.0
F8 = jnp.float8_e4m3fn


def _rows_lt(n, base_rows, shape):
    rows = base_rows + jax.lax.broadcasted_iota(jnp.int32, shape, 0)
    return rows < n


def _support0_body(n, x_ref, w_ref, o_ref):
    r = pl.program_id(0)
    s = jnp.dot(x_ref[...].astype(jnp.bfloat16), w_ref[...],
                preferred_element_type=jnp.float32)
    s = jnp.where(_rows_lt(n, r * o_ref.shape[0], s.shape), s, 0.0)
    o_ref[...] = s.astype(jnp.bfloat16)


def _epilogue(acc, b, w, rowmask):
    h = jnp.maximum(acc + b, 0.0)
    s2 = jnp.dot(h.astype(jnp.bfloat16), w,
                 preferred_element_type=jnp.float32)
    return jnp.where(rowmask, s2, 0.0).astype(jnp.bfloat16)


def _layer1_body(n, adj_ref, s_ref, w_ref, b_ref, ones_ref,
                 aq_ref, o_ref, ocrow_ref):
    r = pl.program_id(0)
    a = adj_ref[...]
    rows = _rows_lt(n, r * BR1, a.shape)
    cols = jax.lax.broadcasted_iota(jnp.int32, a.shape, 1) < n
    am = jnp.where(rows & cols, a, 0.0)
    aq = (am * ASCALE).astype(F8)
    aq_ref[...] = aq
    a16 = aq.astype(jnp.bfloat16)
    rs = jnp.dot(a16, ones_ref[...], preferred_element_type=jnp.float32
                 )[:, 0:1]
    ocrow_ref[...] = jnp.where(rs > 0.0, 1.0 / rs, 0.0)
    acc = jnp.dot(am.astype(jnp.bfloat16), s_ref[...],
                  preferred_element_type=jnp.float32)
    o_ref[...] = _epilogue(acc, b_ref[...], w_ref[...],
                           _rows_lt(n, r * BR1, (BR1, F)))


def _stack_body(n, nl, s2_ref, aq_ref, crow_ref, w_ref, b_ref, o_ref,
                s_scr):
    l = pl.program_id(0)
    r = pl.program_id(1)

    @pl.when(jnp.logical_and(l == 0, r == 0))
    def _():
        s_scr[0] = s2_ref[...]

    p = jax.lax.rem(l, 2)
    a16 = aq_ref[...].astype(jnp.bfloat16)
    acc = jnp.dot(a16, s_scr[p], preferred_element_type=jnp.float32)
    acc = acc * crow_ref[...]

    @pl.when(l < nl - 1)
    def _():
        s2n = _epilogue(acc, b_ref[0], w_ref[0],
                        _rows_lt(n, r * BR, (BR, F)))
        s_scr[1 - p, pl.ds(r * BR, BR), :] = s2n

    @pl.when(l == nl - 1)
    def _():
        o_ref[...] = acc + b_ref[0]


def _cparams():
    return pltpu.CompilerParams(dimension_semantics=("arbitrary",))


def kernel(x, adj, W1, b1, W2, b2, W3, b3, W4, b4, W5, b5, W6, b6):
    n = x.shape[0]
    np_ = ((n + _PAD - 1) // _PAD) * _PAD
    f32 = jnp.float32
    bf16 = jnp.bfloat16
    gr1, gr = np_ // BR1, np_ // BR
    w16 = [w.astype(bf16) for w in (W1, W2, W3, W4, W5, W6)]
    bs = [b.reshape(1, F) for b in (b1, b2, b3, b4, b5, b6)]

    full_s = pl.BlockSpec((np_, F), lambda r: (0, 0))
    full_w = pl.BlockSpec((F, F), lambda r: (0, 0))
    full_b = pl.BlockSpec((1, F), lambda r: (0, 0))

    s = pl.pallas_call(
        functools.partial(_support0_body, n),
        grid=(gr,),
        in_specs=[pl.BlockSpec((BR, F), lambda r: (r, 0)), full_w],
        out_specs=pl.BlockSpec((BR, F), lambda r: (r, 0)),
        out_shape=jax.ShapeDtypeStruct((np_, F), bf16),
        compiler_params=_cparams(),
    )(x, w16[0])

    ones16 = jnp.ones((np_, F), bf16)
    adjq, s, crow = pl.pallas_call(
        functools.partial(_layer1_body, n),
        grid=(gr1,),
        in_specs=[pl.BlockSpec((BR1, np_), lambda r: (r, 0)),
                  full_s, full_w, full_b, full_s],
        out_specs=[pl.BlockSpec((BR1, np_), lambda r: (r, 0)),
                   pl.BlockSpec((BR1, F), lambda r: (r, 0)),
                   pl.BlockSpec((BR1, 1), lambda r: (r, 0))],
        out_shape=[jax.ShapeDtypeStruct((np_, np_), F8),
                   jax.ShapeDtypeStruct((np_, F), bf16),
                   jax.ShapeDtypeStruct((np_, 1), f32)],
        compiler_params=_cparams(),
    )(adj, s, w16[1], bs[0], ones16)

    nl = 5
    full_s2 = pl.BlockSpec((np_, F), lambda l, r: (0, 0))
    wstack = jnp.stack([w16[2], w16[3], w16[4], w16[5], w16[5]])
    bstack = jnp.stack(bs[1:6])
    out = pl.pallas_call(
        functools.partial(_stack_body, n, nl),
        grid=(nl, gr),
        in_specs=[full_s2,
                  pl.BlockSpec((BR, np_), lambda l, r: (r, 0)),
                  pl.BlockSpec((BR, 1), lambda l, r: (r, 0)),
                  pl.BlockSpec((1, F, F), lambda l, r: (l, 0, 0)),
                  pl.BlockSpec((1, 1, F), lambda l, r: (l, 0, 0))],
        out_specs=pl.BlockSpec(
            (BR, F),
            lambda l, r: (jax.lax.select(l == nl - 1, r, 0), 0)),
        out_shape=jax.ShapeDtypeStruct((np_, F), f32),
        scratch_shapes=[pltpu.VMEM((2, np_, F), bf16)],
        compiler_params=pltpu.CompilerParams(
            dimension_semantics=("arbitrary", "arbitrary")),
    )(s, adjq, crow, wstack, bstack)

    return out[:n]


# speedup vs baseline: 1.0755x; 1.0009x over previous
import functools

import jax
import jax.numpy as jnp
from jax.experimental import pallas as pl
from jax.experimental.pallas import tpu as pltpu

F = 128
BR1 = 320
BR = 640
_PAD = 640
ASCALE = 8192.0
F8 = jnp.float8_e4m3fn


def _rows_lt(n, base_rows, shape):
    rows = base_rows + jax.lax.broadcasted_iota(jnp.int32, shape, 0)
    return rows < n


def _epilogue(acc, b, w, rowmask):
    h = jnp.maximum(acc + b, 0.0)
    s2 = jnp.dot(h.astype(jnp.bfloat16), w,
                 preferred_element_type=jnp.float32)
    return jnp.where(rowmask, s2, 0.0).astype(jnp.bfloat16)


def _layer1_body(n, x_ref, adj_ref, w1_ref, w_ref, b_ref, ones_ref,
                 aq_ref, o_ref, ocrow_ref, s_scr):
    i = pl.program_id(0)

    @pl.when(i == 0)
    def _():
        s = jnp.dot(x_ref[...].astype(jnp.bfloat16), w1_ref[...],
                    preferred_element_type=jnp.float32)
        s = jnp.where(_rows_lt(n, 0, s.shape), s, 0.0)
        s_scr[...] = s.astype(jnp.bfloat16)

    @pl.when(i > 0)
    def _():
        r = i - 1
        a = adj_ref[...]
        rows = _rows_lt(n, r * BR1, a.shape)
        cols = jax.lax.broadcasted_iota(jnp.int32, a.shape, 1) < n
        am = jnp.where(rows & cols, a, 0.0)
        aq = (am * ASCALE).astype(F8)
        aq_ref[...] = aq
        a16 = aq.astype(jnp.bfloat16)
        rs = jnp.dot(a16, ones_ref[...], preferred_element_type=jnp.float32
                     )[:, 0:1]
        ocrow_ref[...] = jnp.where(rs > 0.0, 1.0 / rs, 0.0)
        acc = jnp.dot(am.astype(jnp.bfloat16), s_scr[...],
                      preferred_element_type=jnp.float32)
        o_ref[...] = _epilogue(acc, b_ref[...], w_ref[...],
                               _rows_lt(n, r * BR1, (BR1, F)))


def _stack_body(n, nl, s2_ref, aq_ref, crow_ref, w_ref, b_ref, o_ref,
                s_scr):
    l = pl.program_id(0)
    r = pl.program_id(1)

    @pl.when(jnp.logical_and(l == 0, r == 0))
    def _():
        s_scr[0] = s2_ref[...]

    p = jax.lax.rem(l, 2)
    a16 = aq_ref[...].astype(jnp.bfloat16)
    acc = jnp.dot(a16, s_scr[p], preferred_element_type=jnp.float32)
    acc = acc * crow_ref[...]

    @pl.when(l < nl - 1)
    def _():
        s2n = _epilogue(acc, b_ref[0], w_ref[0],
                        _rows_lt(n, r * BR, (BR, F)))
        s_scr[1 - p, pl.ds(r * BR, BR), :] = s2n

    @pl.when(l == nl - 1)
    def _():
        o_ref[...] = acc + b_ref[0]


def _cparams():
    return pltpu.CompilerParams(dimension_semantics=("arbitrary",))


def kernel(x, adj, W1, b1, W2, b2, W3, b3, W4, b4, W5, b5, W6, b6):
    n = x.shape[0]
    np_ = ((n + _PAD - 1) // _PAD) * _PAD
    f32 = jnp.float32
    bf16 = jnp.bfloat16
    gr1, gr = np_ // BR1, np_ // BR
    w16 = [w.astype(bf16) for w in (W1, W2, W3, W4, W5, W6)]
    bs = [b.reshape(1, F) for b in (b1, b2, b3, b4, b5, b6)]

    full_s = pl.BlockSpec((np_, F), lambda r: (0, 0))
    full_w = pl.BlockSpec((F, F), lambda r: (0, 0))
    full_b = pl.BlockSpec((1, F), lambda r: (0, 0))

    ones16 = jnp.ones((np_, F), bf16)
    strip1 = lambda i: (jnp.maximum(i - 1, 0), 0)
    adjq, s, crow = pl.pallas_call(
        functools.partial(_layer1_body, n),
        grid=(gr1 + 1,),
        in_specs=[full_s,
                  pl.BlockSpec((BR1, np_), strip1),
                  full_w, full_w, full_b, full_s],
        out_specs=[pl.BlockSpec((BR1, np_), strip1),
                   pl.BlockSpec((BR1, F), strip1),
                   pl.BlockSpec((BR1, 1), strip1)],
        out_shape=[jax.ShapeDtypeStruct((np_, np_), F8),
                   jax.ShapeDtypeStruct((np_, F), bf16),
                   jax.ShapeDtypeStruct((np_, 1), f32)],
        scratch_shapes=[pltpu.VMEM((np_, F), bf16)],
        compiler_params=_cparams(),
    )(x, adj, w16[0], w16[1], bs[0], ones16)

    nl = 5
    full_s2 = pl.BlockSpec((np_, F), lambda l, r: (0, 0))
    wstack = jnp.stack([w16[2], w16[3], w16[4], w16[5], w16[5]])
    bstack = jnp.stack(bs[1:6])
    out = pl.pallas_call(
        functools.partial(_stack_body, n, nl),
        grid=(nl, gr),
        in_specs=[full_s2,
                  pl.BlockSpec((BR, np_), lambda l, r: (r, 0)),
                  pl.BlockSpec((BR, 1), lambda l, r: (r, 0)),
                  pl.BlockSpec((1, F, F), lambda l, r: (l, 0, 0)),
                  pl.BlockSpec((1, 1, F), lambda l, r: (l, 0, 0))],
        out_specs=pl.BlockSpec(
            (BR, F),
            lambda l, r: (jax.lax.select(l == nl - 1, r, 0), 0)),
        out_shape=jax.ShapeDtypeStruct((np_, F), f32),
        scratch_shapes=[pltpu.VMEM((2, np_, F), bf16)],
        compiler_params=pltpu.CompilerParams(
            dimension_semantics=("arbitrary", "arbitrary")),
    )(s, adjq, crow, wstack, bstack)

    return out[:n]


# speedup vs baseline: 1.0938x; 1.0171x over previous
import functools

import jax
import jax.numpy as jnp
from jax.experimental import pallas as pl
from jax.experimental.pallas import tpu as pltpu

F = 128
BR1 = 320
BR = 1024
_PAD = 5120
ASCALE = 8192.0
F8 = jnp.float8_e4m3fn


def _rows_lt(n, base_rows, shape):
    rows = base_rows + jax.lax.broadcasted_iota(jnp.int32, shape, 0)
    return rows < n


def _epilogue(acc, b, w, rowmask):
    h = jnp.maximum(acc + b, 0.0)
    s2 = jnp.dot(h.astype(jnp.bfloat16), w,
                 preferred_element_type=jnp.float32)
    return jnp.where(rowmask, s2, 0.0).astype(jnp.bfloat16)


def _layer1_body(n, x_ref, adj_ref, w1_ref, w_ref, b_ref, ones_ref,
                 aq_ref, o_ref, ocrow_ref, s_scr):
    i = pl.program_id(0)

    @pl.when(i == 0)
    def _():
        s = jnp.dot(x_ref[...].astype(jnp.bfloat16), w1_ref[...],
                    preferred_element_type=jnp.float32)
        s = jnp.where(_rows_lt(n, 0, s.shape), s, 0.0)
        s_scr[...] = s.astype(jnp.bfloat16)

    @pl.when(i > 0)
    def _():
        r = i - 1
        a = adj_ref[...]
        rows = _rows_lt(n, r * BR1, a.shape)
        cols = jax.lax.broadcasted_iota(jnp.int32, a.shape, 1) < n
        am = jnp.where(rows & cols, a, 0.0)
        aq = (am * ASCALE).astype(F8)
        aq_ref[...] = aq
        a16 = aq.astype(jnp.bfloat16)
        rs = jnp.dot(a16, ones_ref[...], preferred_element_type=jnp.float32
                     )[:, 0:1]
        ocrow_ref[...] = jnp.where(rs > 0.0, 1.0 / rs, 0.0)
        acc = jnp.dot(am.astype(jnp.bfloat16), s_scr[...],
                      preferred_element_type=jnp.float32)
        o_ref[...] = _epilogue(acc, b_ref[...], w_ref[...],
                               _rows_lt(n, r * BR1, (BR1, F)))


def _stack_body(n, nl, s2_ref, aq_ref, crow_ref, w_ref, b_ref, o_ref,
                s_scr):
    l = pl.program_id(0)
    r = pl.program_id(1)

    @pl.when(jnp.logical_and(l == 0, r == 0))
    def _():
        s_scr[0] = s2_ref[...]

    p = jax.lax.rem(l, 2)
    a16 = aq_ref[...].astype(jnp.bfloat16)
    acc = jnp.dot(a16, s_scr[p], preferred_element_type=jnp.float32)
    acc = acc * crow_ref[...]

    @pl.when(l < nl - 1)
    def _():
        s2n = _epilogue(acc, b_ref[0], w_ref[0],
                        _rows_lt(n, r * BR, (BR, F)))
        s_scr[1 - p, pl.ds(r * BR, BR), :] = s2n

    @pl.when(l == nl - 1)
    def _():
        o_ref[...] = acc + b_ref[0]


def _cparams():
    return pltpu.CompilerParams(dimension_semantics=("arbitrary",))


def kernel(x, adj, W1, b1, W2, b2, W3, b3, W4, b4, W5, b5, W6, b6):
    n = x.shape[0]
    np_ = ((n + _PAD - 1) // _PAD) * _PAD
    f32 = jnp.float32
    bf16 = jnp.bfloat16
    gr1, gr = np_ // BR1, np_ // BR
    w16 = [w.astype(bf16) for w in (W1, W2, W3, W4, W5, W6)]
    bs = [b.reshape(1, F) for b in (b1, b2, b3, b4, b5, b6)]

    full_s = pl.BlockSpec((np_, F), lambda r: (0, 0))
    full_w = pl.BlockSpec((F, F), lambda r: (0, 0))
    full_b = pl.BlockSpec((1, F), lambda r: (0, 0))

    ones16 = jnp.ones((np_, F), bf16)
    strip1 = lambda i: (jnp.maximum(i - 1, 0), 0)
    adjq, s, crow = pl.pallas_call(
        functools.partial(_layer1_body, n),
        grid=(gr1 + 1,),
        in_specs=[full_s,
                  pl.BlockSpec((BR1, np_), strip1),
                  full_w, full_w, full_b, full_s],
        out_specs=[pl.BlockSpec((BR1, np_), strip1),
                   pl.BlockSpec((BR1, F), strip1),
                   pl.BlockSpec((BR1, 1), strip1)],
        out_shape=[jax.ShapeDtypeStruct((np_, np_), F8),
                   jax.ShapeDtypeStruct((np_, F), bf16),
                   jax.ShapeDtypeStruct((np_, 1), f32)],
        scratch_shapes=[pltpu.VMEM((np_, F), bf16)],
        compiler_params=_cparams(),
    )(x, adj, w16[0], w16[1], bs[0], ones16)

    nl = 5
    full_s2 = pl.BlockSpec((np_, F), lambda l, r: (0, 0))
    wstack = jnp.stack([w16[2], w16[3], w16[4], w16[5], w16[5]])
    bstack = jnp.stack(bs[1:6])
    out = pl.pallas_call(
        functools.partial(_stack_body, n, nl),
        grid=(nl, gr),
        in_specs=[full_s2,
                  pl.BlockSpec((BR, np_), lambda l, r: (r, 0)),
                  pl.BlockSpec((BR, 1), lambda l, r: (r, 0)),
                  pl.BlockSpec((1, F, F), lambda l, r: (l, 0, 0)),
                  pl.BlockSpec((1, 1, F), lambda l, r: (l, 0, 0))],
        out_specs=pl.BlockSpec(
            (BR, F),
            lambda l, r: (jax.lax.select(l == nl - 1, r, 0), 0)),
        out_shape=jax.ShapeDtypeStruct((np_, F), f32),
        scratch_shapes=[pltpu.VMEM((2, np_, F), bf16)],
        compiler_params=pltpu.CompilerParams(
            dimension_semantics=("arbitrary", "arbitrary")),
    )(s, adjq, crow, wstack, bstack)

    return out[:n]
